# Initial kernel scaffold; baseline (speedup 1.0000x reference)
import jax
import jax.numpy as jnp
from jax.experimental import pallas as pl


def kernel(x, edge_index, batch, global_features, s0Wl, s0bl, s0Wr, s0g, s0b, sWl, sbl, sWr, sg, sb, m0W, m0b, m0g, m0be, mW, mb, mg, mbe, hW, hb):
    raise NotImplementedError("write your pallas kernel here")


# speedup vs baseline: 4.6602x; 4.6602x over previous
import functools

import jax
import jax.numpy as jnp
from jax import lax
from jax.experimental import pallas as pl
from jax.experimental.pallas import tpu as pltpu
from jax.experimental.pallas import tpu_sc as plsc

N = 10000
E = 160000
G = 64
DIN = 261
C = 64
LC = 32
GD = 24

BN = 2000
NB = N // BN
CH = 128
NWORK = 32
NPAD = 10240
RPT = NPAD // 16
EPAD = 163840
CPW = EPAD // (NWORK * CH)
EPW = CPW * CH

_SQRT_HALF = 0.7071067811865476


def _gelu_tc(x):
    return 0.5 * x * (1.0 + lax.erf(x * _SQRT_HALF))


def _ln_tc(x, g, b):
    mu = jnp.mean(x, axis=-1, keepdims=True)
    v = jnp.mean((x - mu) ** 2, axis=-1, keepdims=True)
    return (x - mu) * lax.rsqrt(v + 1e-5) * g + b


def _pre_body(x_ref, wl_ref, wr_ref, bl_ref, p_ref, r_ref):
    xb = x_ref[...]
    p_ref[...] = jnp.dot(xb, wl_ref[...], preferred_element_type=jnp.float32)
    r_ref[...] = (jnp.dot(xb, wr_ref[...], preferred_element_type=jnp.float32)
                  + bl_ref[...])


def _tc_pre(x, wl, wr, bl):
    return pl.pallas_call(
        _pre_body,
        grid=(NB,),
        in_specs=[
            pl.BlockSpec((BN, DIN), lambda i: (i, 0)),
            pl.BlockSpec((DIN, C), lambda i: (0, 0)),
            pl.BlockSpec((DIN, C), lambda i: (0, 0)),
            pl.BlockSpec((1, C), lambda i: (0, 0)),
        ],
        out_specs=[
            pl.BlockSpec((BN, C), lambda i: (i, 0)),
            pl.BlockSpec((BN, C), lambda i: (i, 0)),
        ],
        out_shape=[
            jax.ShapeDtypeStruct((N, C), jnp.float32),
            jax.ShapeDtypeStruct((N, C), jnp.float32),
        ],
    )(x, wl, wr, bl)


def _mid1_body(a0_ref, a1_ref, c0_ref, c1_ref, r_ref, g_ref, b_ref,
               wl_ref, wr_ref, bl_ref, p_ref, rn_ref, inv_ref):
    cnt = c0_ref[:, :1] + c1_ref[:, :1]
    inv = 1.0 / jnp.maximum(cnt, 1.0)
    f = (a0_ref[...] + a1_ref[...]) * inv + r_ref[...]
    h = _ln_tc(_gelu_tc(f), g_ref[...], b_ref[...])
    p_ref[...] = jnp.dot(h, wl_ref[...], preferred_element_type=jnp.float32)
    rn_ref[...] = (jnp.dot(h, wr_ref[...], preferred_element_type=jnp.float32)
                   + bl_ref[...])
    inv_ref[...] = inv


def _tc_mid1(a0, a1, c0, c1, r, g, b, wl, wr, bl):
    return pl.pallas_call(
        _mid1_body,
        grid=(NB,),
        in_specs=[
            pl.BlockSpec((BN, C), lambda i: (i, 0)),
            pl.BlockSpec((BN, C), lambda i: (i, 0)),
            pl.BlockSpec((BN, 16), lambda i: (i, 0)),
            pl.BlockSpec((BN, 16), lambda i: (i, 0)),
            pl.BlockSpec((BN, C), lambda i: (i, 0)),
            pl.BlockSpec((1, C), lambda i: (0, 0)),
            pl.BlockSpec((1, C), lambda i: (0, 0)),
            pl.BlockSpec((C, C), lambda i: (0, 0)),
            pl.BlockSpec((C, C), lambda i: (0, 0)),
            pl.BlockSpec((1, C), lambda i: (0, 0)),
        ],
        out_specs=[
            pl.BlockSpec((BN, C), lambda i: (i, 0)),
            pl.BlockSpec((BN, C), lambda i: (i, 0)),
            pl.BlockSpec((BN, 1), lambda i: (i, 0)),
        ],
        out_shape=[
            jax.ShapeDtypeStruct((N, C), jnp.float32),
            jax.ShapeDtypeStruct((N, C), jnp.float32),
            jax.ShapeDtypeStruct((N, 1), jnp.float32),
        ],
    )(a0, a1, c0, c1, r, g, b, wl, wr, bl)


def _mid_body(a0_ref, a1_ref, inv_ref, r_ref, g_ref, b_ref,
              wl_ref, wr_ref, bl_ref, p_ref, rn_ref):
    f = (a0_ref[...] + a1_ref[...]) * inv_ref[...] + r_ref[...]
    h = _ln_tc(_gelu_tc(f), g_ref[...], b_ref[...])
    p_ref[...] = jnp.dot(h, wl_ref[...], preferred_element_type=jnp.float32)
    rn_ref[...] = (jnp.dot(h, wr_ref[...], preferred_element_type=jnp.float32)
                   + bl_ref[...])


def _tc_mid(a0, a1, inv, r, g, b, wl, wr, bl):
    return pl.pallas_call(
        _mid_body,
        grid=(NB,),
        in_specs=[
            pl.BlockSpec((BN, C), lambda i: (i, 0)),
            pl.BlockSpec((BN, C), lambda i: (i, 0)),
            pl.BlockSpec((BN, 1), lambda i: (i, 0)),
            pl.BlockSpec((BN, C), lambda i: (i, 0)),
            pl.BlockSpec((1, C), lambda i: (0, 0)),
            pl.BlockSpec((1, C), lambda i: (0, 0)),
            pl.BlockSpec((C, C), lambda i: (0, 0)),
            pl.BlockSpec((C, C), lambda i: (0, 0)),
            pl.BlockSpec((1, C), lambda i: (0, 0)),
        ],
        out_specs=[
            pl.BlockSpec((BN, C), lambda i: (i, 0)),
            pl.BlockSpec((BN, C), lambda i: (i, 0)),
        ],
        out_shape=[
            jax.ShapeDtypeStruct((N, C), jnp.float32),
            jax.ShapeDtypeStruct((N, C), jnp.float32),
        ],
    )(a0, a1, inv, r, g, b, wl, wr, bl)


def _final_body(a0_ref, a1_ref, inv_ref, r_ref, g_ref, b_ref, bat_ref, gf_ref,
                m0wp_ref, m0wg_ref, m0b_ref, m0g_ref, m0be_ref,
                mw_ref, mb_ref, mg_ref, mbe_ref, hw_ref, hb_ref,
                out_ref, pool_acc, cnt_acc):
    pid = pl.program_id(0)
    f = (a0_ref[...] + a1_ref[...]) * inv_ref[...] + r_ref[...]
    h = _ln_tc(_gelu_tc(f), g_ref[...], b_ref[...])
    onehot = (bat_ref[...] == lax.broadcasted_iota(jnp.int32, (BN, G), 1)
              ).astype(jnp.float32)
    psum = lax.dot_general(onehot, h, (((0,), (0,)), ((), ())),
                           preferred_element_type=jnp.float32)
    csum = lax.dot_general(onehot, jnp.ones((BN, 1), jnp.float32),
                           (((0,), (0,)), ((), ())),
                           preferred_element_type=jnp.float32)

    @pl.when(pid == 0)
    def _():
        pool_acc[...] = psum
        cnt_acc[...] = csum

    @pl.when(pid > 0)
    def _():
        pool_acc[...] += psum
        cnt_acc[...] += csum

    @pl.when(pid == NB - 1)
    def _():
        pool = pool_acc[...] / jnp.maximum(cnt_acc[...], 1.0)
        f0 = (jnp.dot(pool, m0wp_ref[...], preferred_element_type=jnp.float32)
              + jnp.dot(gf_ref[...], m0wg_ref[...],
                        preferred_element_type=jnp.float32)
              + m0b_ref[...])
        fl = _ln_tc(_gelu_tc(f0), m0g_ref[...], m0be_ref[...])
        for i in range(3):
            t = jnp.dot(fl, mw_ref[i], preferred_element_type=jnp.float32) + mb_ref[i]
            fl = _ln_tc(_gelu_tc(t), mg_ref[i], mbe_ref[i]) + fl
        out_ref[...] = (jnp.dot(fl, hw_ref[...],
                                preferred_element_type=jnp.float32)
                        + hb_ref[...])


def _tc_final(a0, a1, inv, r, g, b, bat2, gf,
              m0wp, m0wg, m0b, m0g, m0be, mw, mb, mg, mbe, hw, hb):
    return pl.pallas_call(
        _final_body,
        grid=(NB,),
        in_specs=[
            pl.BlockSpec((BN, C), lambda i: (i, 0)),
            pl.BlockSpec((BN, C), lambda i: (i, 0)),
            pl.BlockSpec((BN, 1), lambda i: (i, 0)),
            pl.BlockSpec((BN, C), lambda i: (i, 0)),
            pl.BlockSpec((1, C), lambda i: (0, 0)),
            pl.BlockSpec((1, C), lambda i: (0, 0)),
            pl.BlockSpec((BN, 1), lambda i: (i, 0)),
            pl.BlockSpec((G, GD), lambda i: (0, 0)),
            pl.BlockSpec((C, LC), lambda i: (0, 0)),
            pl.BlockSpec((GD, LC), lambda i: (0, 0)),
            pl.BlockSpec((1, LC), lambda i: (0, 0)),
            pl.BlockSpec((1, LC), lambda i: (0, 0)),
            pl.BlockSpec((1, LC), lambda i: (0, 0)),
            pl.BlockSpec((3, LC, LC), lambda i: (0, 0, 0)),
            pl.BlockSpec((3, 1, LC), lambda i: (0, 0, 0)),
            pl.BlockSpec((3, 1, LC), lambda i: (0, 0, 0)),
            pl.BlockSpec((3, 1, LC), lambda i: (0, 0, 0)),
            pl.BlockSpec((LC, 1), lambda i: (0, 0)),
            pl.BlockSpec((1, 1), lambda i: (0, 0)),
        ],
        out_specs=pl.BlockSpec((G, 1), lambda i: (0, 0)),
        out_shape=jax.ShapeDtypeStruct((G, 1), jnp.float32),
        scratch_shapes=[
            pltpu.VMEM((G, C), jnp.float32),
            pltpu.VMEM((G, 1), jnp.float32),
        ],
    )(a0, a1, inv, r, g, b, bat2, gf,
      m0wp, m0wg, m0b, m0g, m0be, mw, mb, mg, mbe, hw, hb)


@functools.cache
def _sc_mesh():
    return plsc.VectorSubcoreMesh(
        core_axis_name="c", subcore_axis_name="s", num_cores=2,
        num_subcores=16)


def _sc_agg_body(p_hbm, srcg, dstg, zero_hbm, out0, out1, sidx, didx, rows,
                 aggs, sem):
    cid = lax.axis_index("c")
    sid = lax.axis_index("s")
    wid = sid * 2 + cid
    row0 = pl.multiple_of(sid * RPT, 8)
    pltpu.sync_copy(zero_hbm.at[pl.ds(row0, RPT)], aggs.at[pl.ds(row0, RPT)])
    crow = pl.multiple_of(wid * CPW, 8)
    pltpu.sync_copy(srcg.at[pl.ds(crow, CPW)], sidx)
    pltpu.sync_copy(dstg.at[pl.ds(crow, CPW)], didx)
    plsc.subcore_barrier()

    def body(k, carry):
        pltpu.async_copy(p_hbm.at[sidx.at[k]], rows, sem).wait()
        pltpu.sync_copy(rows, aggs.at[didx.at[k]], add=True)
        return carry

    lax.fori_loop(0, CPW, body, 0)
    plsc.subcore_barrier()

    @pl.when(cid == 0)
    def _():
        pltpu.sync_copy(aggs.at[pl.ds(row0, RPT)], out0.at[pl.ds(row0, RPT)])

    @pl.when(cid == 1)
    def _():
        pltpu.sync_copy(aggs.at[pl.ds(row0, RPT)], out1.at[pl.ds(row0, RPT)])


@functools.cache
def _sc_agg_kernel():
    return pl.kernel(
        _sc_agg_body,
        out_type=[jax.ShapeDtypeStruct((NPAD, C), jnp.float32),
                  jax.ShapeDtypeStruct((NPAD, C), jnp.float32)],
        mesh=_sc_mesh(),
        scratch_types=[
            pltpu.VMEM((CPW, CH), jnp.int32),
            pltpu.VMEM((CPW, CH), jnp.int32),
            pltpu.VMEM((CH, C), jnp.float32),
            pltpu.VMEM_SHARED((NPAD, C), jnp.float32),
            pltpu.SemaphoreType.DMA,
        ],
        compiler_params=pltpu.CompilerParams(use_tc_tiling_on_sc=False),
    )


def _sc_agg(p, srcg, dstg, zero64):
    return _sc_agg_kernel()(p, srcg, dstg, zero64)


def _sc_cnt_body(dstg, ones_hbm, zero_hbm, out0, out1, didx, ones_v, cnts):
    cid = lax.axis_index("c")
    sid = lax.axis_index("s")
    wid = sid * 2 + cid
    row0 = pl.multiple_of(sid * RPT, 8)
    pltpu.sync_copy(ones_hbm, ones_v)
    pltpu.sync_copy(zero_hbm.at[pl.ds(row0, RPT)], cnts.at[pl.ds(row0, RPT)])
    crow = pl.multiple_of(wid * CPW, 8)
    pltpu.sync_copy(dstg.at[pl.ds(crow, CPW)], didx)
    plsc.subcore_barrier()

    def body(k, carry):
        pltpu.sync_copy(ones_v, cnts.at[didx.at[k]], add=True)
        return carry

    lax.fori_loop(0, CPW, body, 0)
    plsc.subcore_barrier()

    @pl.when(cid == 0)
    def _():
        pltpu.sync_copy(cnts.at[pl.ds(row0, RPT)], out0.at[pl.ds(row0, RPT)])

    @pl.when(cid == 1)
    def _():
        pltpu.sync_copy(cnts.at[pl.ds(row0, RPT)], out1.at[pl.ds(row0, RPT)])


@functools.cache
def _sc_cnt_kernel():
    return pl.kernel(
        _sc_cnt_body,
        out_type=[jax.ShapeDtypeStruct((NPAD, 16), jnp.float32),
                  jax.ShapeDtypeStruct((NPAD, 16), jnp.float32)],
        mesh=_sc_mesh(),
        scratch_types=[
            pltpu.VMEM((CPW, CH), jnp.int32),
            pltpu.VMEM((CH, 16), jnp.float32),
            pltpu.VMEM_SHARED((NPAD, 16), jnp.float32),
        ],
        compiler_params=pltpu.CompilerParams(use_tc_tiling_on_sc=False),
    )


def _sc_cnt(dstg, ones16, zero16):
    return _sc_cnt_kernel()(dstg, ones16, zero16)


def kernel(x, edge_index, batch, global_features, s0Wl, s0bl, s0Wr, s0g, s0b,
           sWl, sbl, sWr, sg, sb, m0W, m0b, m0g, m0be, mW, mb, mg, mbe,
           hW, hb):
    epad = EPAD - E
    srcg = jnp.concatenate(
        [edge_index[0], jnp.zeros((epad,), jnp.int32)]).reshape(EPAD // CH, CH)
    dstg = jnp.concatenate(
        [edge_index[1], jnp.full((epad,), N, jnp.int32)]).reshape(EPAD // CH, CH)
    zero64 = jnp.zeros((NPAD, C), jnp.float32)
    zero16 = jnp.zeros((NPAD, 16), jnp.float32)
    ones16 = jnp.ones((CH, 16), jnp.float32)
    bat2 = batch.reshape(N, 1)

    r1 = lambda a: a.reshape(1, -1)
    r3 = lambda a: a.reshape(a.shape[0], 1, a.shape[1])

    c0, c1 = _sc_cnt(dstg, ones16, zero16)
    p, r = _tc_pre(x, s0Wl, s0Wr, r1(s0bl))

    a0, a1 = _sc_agg(p, srcg, dstg, zero64)
    p, r, inv = _tc_mid1(a0, a1, c0, c1, r, r1(s0g), r1(s0b),
                         sWl[0], sWr[0], r1(sbl[0]))

    for i in range(1, 6):
        a0, a1 = _sc_agg(p, srcg, dstg, zero64)
        p, r = _tc_mid(a0, a1, inv, r, r1(sg[i - 1]), r1(sb[i - 1]),
                       sWl[i], sWr[i], r1(sbl[i]))

    a0, a1 = _sc_agg(p, srcg, dstg, zero64)
    out = _tc_final(a0, a1, inv, r, r1(sg[5]), r1(sb[5]), bat2,
                    global_features, m0W[:C], m0W[C:], r1(m0b), r1(m0g),
                    r1(m0be), mW, r3(mb), r3(mg), r3(mbe), hW, r1(hb))
    return out
